# Initial kernel scaffold; baseline (speedup 1.0000x reference)
import jax
import jax.numpy as jnp
from jax.experimental import pallas as pl


def kernel(x, table, pos):
    raise NotImplementedError("write your pallas kernel here")


# speedup vs baseline: 1.4243x; 1.4243x over previous
import jax
import jax.numpy as jnp
from jax import lax
from jax.experimental import pallas as pl
from jax.experimental.pallas import tpu as pltpu
from jax.experimental.pallas import tpu_sc as plsc

D = 32
L = 200
B = 4096
N = B * L
NC, NS = 2, 16
NW = NC * NS
PER_W = N // NW
CHUNK = 1600
NCH = PER_W // CHUNK
REP = CHUNK // L
SUB = 100
NSUB = CHUNK // SUB
LANES = 16


def _body(x_hbm, table_hbm, pos_hbm, out_hbm, idx_v, dest_v, pos_v, sem):
    wid = lax.axis_index("s") * NC + lax.axis_index("c")
    base_w = wid * PER_W
    pltpu.sync_copy(pos_hbm, pos_v)

    def chunk_body(c, _):
        base = pl.multiple_of(base_w + c * CHUNK, CHUNK)
        pltpu.sync_copy(x_hbm.at[pl.ds(pl.multiple_of(base // SUB, NSUB), NSUB)], idx_v)
        copies = [
            pltpu.async_copy(
                table_hbm.at[idx_v.at[j]],
                dest_v.at[pl.ds(j * SUB, SUB)],
                sem,
            )
            for j in range(NSUB)
        ]
        for cp in copies:
            cp.wait()

        def add_l(l, _):
            p0 = pos_v[l, pl.ds(0, LANES)]
            p1 = pos_v[l, pl.ds(LANES, LANES)]
            for t in range(REP):
                r = t * L + l
                dest_v[r, pl.ds(0, LANES)] += p0
                dest_v[r, pl.ds(LANES, LANES)] += p1
            return 0

        lax.fori_loop(0, L, add_l, 0, unroll=False)
        pltpu.sync_copy(dest_v, out_hbm.at[pl.ds(base, CHUNK)])
        return 0

    lax.fori_loop(0, NCH, chunk_body, 0, unroll=False)


_mesh = plsc.VectorSubcoreMesh(core_axis_name="c", subcore_axis_name="s")

_embed = pl.kernel(
    _body,
    out_type=jax.ShapeDtypeStruct((N, D), jnp.float32),
    mesh=_mesh,
    scratch_types=[
        pltpu.VMEM((NSUB, SUB), jnp.int32),
        pltpu.VMEM((CHUNK, D), jnp.float32),
        pltpu.VMEM((L, D), jnp.float32),
        pltpu.SemaphoreType.DMA,
    ],
    compiler_params=pltpu.CompilerParams(use_tc_tiling_on_sc=False),
)


@jax.jit
def _run(x, table, pos):
    x2d = x.reshape(N // SUB, SUB).astype(jnp.int32)
    out = _embed(x2d, table, pos)
    return out.reshape(B, L, D)


def kernel(x, table, pos):
    return _run(x, table, pos)


# speedup vs baseline: 1.4297x; 1.0038x over previous
import jax
import jax.numpy as jnp
from jax import lax
from jax.experimental import pallas as pl
from jax.experimental.pallas import tpu as pltpu
from jax.experimental.pallas import tpu_sc as plsc

D = 32
L = 200
B = 4096
V = 1000000
NC, NS = 2, 16
NW = NC * NS
BPW = B // NW
RPC = 8
CHUNK = RPC * L
NCH = BPW // RPC
SUB = 80
NSUB = CHUNK // SUB
LANES = 16
_COLS = [k * LANES for k in range(L // LANES)] + [L - LANES]


def _body(x_hbm, table_hbm, pos_hbm, out_hbm, xbuf, idx_v, dest_v, pos_v, sem):
    wid = lax.axis_index("s") * NC + lax.axis_index("c")
    b0 = pl.multiple_of(wid * BPW, BPW)
    pltpu.sync_copy(pos_hbm, pos_v)
    pltpu.sync_copy(x_hbm.at[pl.ds(b0, BPW)], xbuf)

    def chunk_body(ch, _):
        row0 = ch * RPC
        for r in range(RPC):
            for col in _COLS:
                idx_v[pl.ds(r * L + col, LANES)] = xbuf[row0 + r, pl.ds(col, LANES)]
        copies = [
            pltpu.async_copy(
                table_hbm.at[idx_v.at[pl.ds(j * SUB, SUB)]],
                dest_v.at[pl.ds(j * SUB, SUB)],
                sem,
            )
            for j in range(NSUB)
        ]
        for cp in copies:
            cp.wait()

        def add_l(l, _):
            p0 = pos_v[l, pl.ds(0, LANES)]
            p1 = pos_v[l, pl.ds(LANES, LANES)]
            for t in range(RPC):
                r2 = t * L + l
                dest_v[r2, pl.ds(0, LANES)] += p0
                dest_v[r2, pl.ds(LANES, LANES)] += p1
            return 0

        lax.fori_loop(0, L, add_l, 0)
        for r in range(RPC):
            pltpu.sync_copy(
                dest_v.at[pl.ds(r * L, L)],
                out_hbm.at[b0 + row0 + r],
            )
        return 0

    lax.fori_loop(0, NCH, chunk_body, 0)


_mesh = plsc.VectorSubcoreMesh(core_axis_name="c", subcore_axis_name="s")

_embed = pl.kernel(
    _body,
    out_type=jax.ShapeDtypeStruct((B, L, D), jnp.float32),
    mesh=_mesh,
    scratch_types=[
        pltpu.VMEM((BPW, L), jnp.int32),
        pltpu.VMEM((CHUNK,), jnp.int32),
        pltpu.VMEM((CHUNK, D), jnp.float32),
        pltpu.VMEM((L, D), jnp.float32),
        pltpu.SemaphoreType.DMA,
    ],
    compiler_params=pltpu.CompilerParams(use_tc_tiling_on_sc=False),
)


@jax.jit
def _run(x, table, pos):
    return _embed(x, table, pos)


def kernel(x, table, pos):
    return _run(x, table, pos)
